# Initial kernel scaffold; baseline (speedup 1.0000x reference)
import jax
import jax.numpy as jnp
from jax.experimental import pallas as pl


def kernel(y_true, y_pred, mask, B):
    raise NotImplementedError("write your pallas kernel here")


# speedup vs baseline: 6.7121x; 6.7121x over previous
import jax
import jax.numpy as jnp
from jax.experimental import pallas as pl
from jax.experimental.pallas import tpu as pltpu

_A = 25
_TILE = 4096


def _blosum_tile(yt_ref, m_ref, yp_ref, bt_ref, num_ref, den_ref):
    i = pl.program_id(0)

    @pl.when(i == 0)
    def _init():
        num_ref[...] = jnp.zeros((1, 1), jnp.float32)
        den_ref[...] = jnp.zeros((1, 1), jnp.float32)

    yt = yt_ref[0, 0, :]
    m = m_ref[0, 0, :]
    yp = yp_ref[...]
    bt = bt_ref[...]

    p = jnp.dot(yp, bt, preferred_element_type=jnp.float32)
    cls = jax.lax.broadcasted_iota(jnp.int32, (_TILE, _A), 1)
    moh = jnp.where(cls == yt.reshape(_TILE, 1),
                    m.reshape(_TILE, 1), 0.0)
    num_ref[...] += jnp.sum(p * moh).reshape(1, 1)
    den_ref[...] += jnp.sum(m).reshape(1, 1)


def kernel(y_true, y_pred, mask, B):
    n = y_true.shape[0] * y_true.shape[1]
    tiles = n // _TILE
    yt = y_true.reshape(tiles, 1, _TILE).astype(jnp.int32)
    m = mask.reshape(tiles, 1, _TILE)
    yp = y_pred.reshape(n, _A)
    bt = B.T

    num, den = pl.pallas_call(
        _blosum_tile,
        grid=(tiles,),
        in_specs=[
            pl.BlockSpec((1, 1, _TILE), lambda i: (i, 0, 0)),
            pl.BlockSpec((1, 1, _TILE), lambda i: (i, 0, 0)),
            pl.BlockSpec((_TILE, _A), lambda i: (i, 0)),
            pl.BlockSpec((_A, _A), lambda i: (0, 0)),
        ],
        out_specs=[
            pl.BlockSpec((1, 1), lambda i: (0, 0)),
            pl.BlockSpec((1, 1), lambda i: (0, 0)),
        ],
        out_shape=[
            jax.ShapeDtypeStruct((1, 1), jnp.float32),
            jax.ShapeDtypeStruct((1, 1), jnp.float32),
        ],
        compiler_params=pltpu.CompilerParams(
            dimension_semantics=("arbitrary",),
        ),
    )(yt, m, yp, bt)
    return num[0, 0] / den[0, 0]


# speedup vs baseline: 7.7927x; 1.1610x over previous
import jax
import jax.numpy as jnp
from jax.experimental import pallas as pl
from jax.experimental.pallas import tpu as pltpu

_A = 25
_TILE = 16384


def _blosum_tile(yt_ref, m_ref, yp_ref, bt_ref, num_ref, den_ref):
    i = pl.program_id(0)

    @pl.when(i == 0)
    def _init():
        num_ref[...] = jnp.zeros((1, 1), jnp.float32)
        den_ref[...] = jnp.zeros((1, 1), jnp.float32)

    yt = yt_ref[0, 0, :]
    m = m_ref[0, 0, :]
    yp = yp_ref[...]
    bt = bt_ref[...]

    p = jnp.dot(yp, bt, preferred_element_type=jnp.float32)
    cls = jax.lax.broadcasted_iota(jnp.int32, (_TILE, _A), 1)
    moh = jnp.where(cls == yt.reshape(_TILE, 1),
                    m.reshape(_TILE, 1), 0.0)
    num_ref[...] += jnp.sum(p * moh).reshape(1, 1)
    den_ref[...] += jnp.sum(m).reshape(1, 1)


def kernel(y_true, y_pred, mask, B):
    n = y_true.shape[0] * y_true.shape[1]
    tiles = n // _TILE
    yt = y_true.reshape(tiles, 1, _TILE).astype(jnp.int32)
    m = mask.reshape(tiles, 1, _TILE)
    yp = y_pred.reshape(n, _A)
    bt = B.T

    num, den = pl.pallas_call(
        _blosum_tile,
        grid=(tiles,),
        in_specs=[
            pl.BlockSpec((1, 1, _TILE), lambda i: (i, 0, 0)),
            pl.BlockSpec((1, 1, _TILE), lambda i: (i, 0, 0)),
            pl.BlockSpec((_TILE, _A), lambda i: (i, 0)),
            pl.BlockSpec((_A, _A), lambda i: (0, 0)),
        ],
        out_specs=[
            pl.BlockSpec((1, 1), lambda i: (0, 0)),
            pl.BlockSpec((1, 1), lambda i: (0, 0)),
        ],
        out_shape=[
            jax.ShapeDtypeStruct((1, 1), jnp.float32),
            jax.ShapeDtypeStruct((1, 1), jnp.float32),
        ],
        compiler_params=pltpu.CompilerParams(
            dimension_semantics=("arbitrary",),
        ),
    )(yt, m, yp, bt)
    return num[0, 0] / den[0, 0]
